# Initial kernel scaffold; baseline (speedup 1.0000x reference)
import jax
import jax.numpy as jnp
from jax.experimental import pallas as pl


def kernel(x, batch, node_attr, edge_src, edge_dst, K_PU, emb, W_na, W_lin1, W_fc1, b_fc1, W_fc2, W_sh, W_msg, W_lin2, W_sc):
    raise NotImplementedError("write your pallas kernel here")


# speedup vs baseline: 2.6712x; 2.6712x over previous
import functools

import jax
import jax.numpy as jnp
import numpy as np
from jax import lax
from jax.experimental import pallas as pl
from jax.experimental.pallas import tpu as pltpu
from jax.experimental.pallas import tpu_sc as plsc

N = 10000
E = 640000
D_IN = 6
D_H = 80
D_CONV = 96
NS = 32
NG = 16
NV = 16
MAX_ATOM = 100
NB = 8
RH = 16
LAYERS = 2
MAX_RADIUS = 2.5
NUM_NEIGHBORS = 64.0
H_STEP = 0.1

SC_CORES = 2
SC_SUBCORES = 16
SC_WORKERS = SC_CORES * SC_SUBCORES
EPW = E // SC_WORKERS

NPAD = 10240
ROWS_PER_SUBCORE = NPAD // SC_SUBCORES

_SC_MESH = dict(core_axis_name="c", subcore_axis_name="s")

F32 = jnp.float32


def _silu(v):
    return v / (1.0 + jnp.exp(-v))


def _sigmoid(v):
    return 1.0 / (1.0 + jnp.exp(-v))


_CP = 4000


def _sc_gather_pos(xpad, edge_src, edge_dst):
    mesh = plsc.VectorSubcoreMesh(**_SC_MESH)

    @functools.partial(
        pl.kernel,
        out_type=(
            jax.ShapeDtypeStruct((E, 8), F32),
            jax.ShapeDtypeStruct((E, 8), F32),
        ),
        mesh=mesh,
        scratch_types=[
            pltpu.VMEM((_CP,), jnp.int32),
            pltpu.VMEM((_CP,), jnp.int32),
            pltpu.VMEM((_CP, 8), F32),
            pltpu.VMEM((_CP, 8), F32),
            pltpu.SemaphoreType.DMA,
            pltpu.SemaphoreType.DMA,
        ],
        compiler_params=pltpu.CompilerParams(use_tc_tiling_on_sc=False),
    )
    def k(xpad_hbm, src_hbm, dst_hbm, outs_hbm, outd_hbm, sidx, didx, sbuf, dbuf, sem1, sem2):
        wid = lax.axis_index("s") * SC_CORES + lax.axis_index("c")
        base = wid * EPW

        def chunk(i, _):
            eb = base + i * _CP
            pltpu.sync_copy(src_hbm.at[pl.ds(eb, _CP)], sidx)
            pltpu.sync_copy(dst_hbm.at[pl.ds(eb, _CP)], didx)
            c1 = pltpu.async_copy(xpad_hbm.at[sidx], sbuf, sem1)
            c2 = pltpu.async_copy(xpad_hbm.at[didx], dbuf, sem2)
            c1.wait()
            c2.wait()
            pltpu.sync_copy(sbuf, outs_hbm.at[pl.ds(eb, _CP)])
            pltpu.sync_copy(dbuf, outd_hbm.at[pl.ds(eb, _CP)])
            return 0

        lax.fori_loop(0, EPW // _CP, chunk, 0)

    return k(xpad, edge_src, edge_dst)


_CS = 200


def _sc_scatter(y3, wsh, edge_src, edge_dst):
    mesh = plsc.VectorSubcoreMesh(**_SC_MESH)

    @functools.partial(
        pl.kernel,
        out_type=jax.ShapeDtypeStruct((SC_CORES, NPAD, D_CONV), F32),
        mesh=mesh,
        scratch_types=[
            pltpu.VMEM((_CS,), jnp.int32),
            pltpu.VMEM((_CS,), jnp.int32),
            pltpu.VMEM((_CS, D_CONV), F32),
            pltpu.VMEM((_CS, D_CONV), F32),
            pltpu.VMEM_SHARED((NPAD, D_CONV), F32),
            pltpu.SemaphoreType.DMA,
        ],
        compiler_params=pltpu.CompilerParams(use_tc_tiling_on_sc=False),
    )
    def k(y3_hbm, wsh_hbm, src_hbm, dst_hbm, out_hbm, sidx, didx, y3b, wshb, aggsh, sem):
        cid = lax.axis_index("c")
        sid = lax.axis_index("s")
        wid = sid * SC_CORES + cid
        base = wid * EPW
        r0 = sid * ROWS_PER_SUBCORE

        def zrow(i, _):
            def zcol(kk, __):
                wshb[i, pl.ds(kk * 16, 16)] = jnp.zeros((16,), F32)
                return 0

            lax.fori_loop(0, D_CONV // 16, zcol, 0)
            return 0

        lax.fori_loop(0, _CS, zrow, 0)
        for off in range(0, ROWS_PER_SUBCORE, _CS):
            m = min(_CS, ROWS_PER_SUBCORE - off)
            pltpu.sync_copy(wshb.at[pl.ds(0, m)], aggsh.at[pl.ds(r0 + off, m)])
        plsc.subcore_barrier()

        def chunk(i, _):
            eb = base + i * _CS
            pltpu.sync_copy(src_hbm.at[pl.ds(eb, _CS)], sidx)
            pltpu.sync_copy(dst_hbm.at[pl.ds(eb, _CS)], didx)
            g = pltpu.async_copy(y3_hbm.at[sidx], y3b, sem)
            pltpu.sync_copy(wsh_hbm.at[pl.ds(eb, _CS)], wshb)
            g.wait()

            def mrow(r, __):
                def mcol(kk, ___):
                    sl = pl.ds(kk * 16, 16)
                    wshb[r, sl] = wshb[r, sl] * y3b[r, sl]
                    return 0

                lax.fori_loop(0, D_CONV // 16, mcol, 0)
                return 0

            lax.fori_loop(0, _CS, mrow, 0)
            pltpu.sync_copy(wshb, aggsh.at[didx], add=True)
            return 0

        lax.fori_loop(0, EPW // _CS, chunk, 0)
        plsc.subcore_barrier()

        for off in range(0, ROWS_PER_SUBCORE, _CS):
            m = min(_CS, ROWS_PER_SUBCORE - off)
            pltpu.sync_copy(aggsh.at[pl.ds(r0 + off, m)], y3b.at[pl.ds(0, m)])
            pltpu.sync_copy(y3b.at[pl.ds(0, m)], out_hbm.at[cid, pl.ds(r0 + off, m)])

    return k(y3, wsh, edge_src, edge_dst)


_BN = 2000
_BE = 4000


def _dot(a, b):
    return jnp.dot(a, b, preferred_element_type=F32)


def _full2d(shape):
    return pl.BlockSpec(shape, lambda i: (0, 0))


def _rows(width, block=_BN):
    return pl.BlockSpec((block, width), lambda i: (i, 0))


def _node_front(y, na, wna, wl1, wmsg, wsc):
    s = _silu(_dot(na, wna))
    ys = y * s
    y3 = _dot(_dot(ys, wl1), wmsg)
    sc = _dot(ys, wsc)
    return y3, sc


def _tc_pre(x, attr, emb, mt, mp, wna, wl1, wmsg, wsc):
    def body(x_r, attr_r, emb_r, mt_r, mp_r, wna_r, wl1_r, wmsg_r, wsc_r,
             y_r, na_r, y3_r, sc_r, xpad_r):
        xb = x_r[...]
        onehot = (attr_r[...] == lax.broadcasted_iota(jnp.int32, (1, MAX_ATOM), 1)).astype(F32)
        na = _dot(onehot, emb_r[...])
        y = _dot(xb, mt_r[...])
        y3, sc = _node_front(y, na, wna_r[...], wl1_r[...], wmsg_r[...], wsc_r[...])
        y_r[...] = y
        na_r[...] = na
        y3_r[...] = y3
        sc_r[...] = sc
        xpad_r[...] = _dot(xb, mp_r[...])

    return pl.pallas_call(
        body,
        grid=(N // _BN,),
        in_specs=[
            _rows(D_IN), _rows(1), _full2d((MAX_ATOM, 32)), _full2d((D_IN, D_H)),
            _full2d((D_IN, 8)), _full2d((32, D_H)), _full2d((D_H, D_H)),
            _full2d((D_H, D_CONV)), _full2d((D_H, D_CONV)),
        ],
        out_specs=[_rows(D_H), _rows(32), _rows(D_CONV), _rows(D_CONV), _rows(8)],
        out_shape=[
            jax.ShapeDtypeStruct((N, D_H), F32),
            jax.ShapeDtypeStruct((N, 32), F32),
            jax.ShapeDtypeStruct((N, D_CONV), F32),
            jax.ShapeDtypeStruct((N, D_CONV), F32),
            jax.ShapeDtypeStruct((N, 8), F32),
        ],
    )(x, attr, emb, mt, mp, wna, wl1, wmsg, wsc)


def _tc_edge(vs, vd, wfc1, bfc1, wfc2, wsh_w):
    def body(vs_r, vd_r, wfc1_r, bfc1_r, wfc2_r, wshw_r, out_r):
        v3 = vs_r[...][:, :3] - vd_r[...][:, :3]
        r2 = jnp.sum(v3 * v3, axis=1, keepdims=True) + 1e-12
        r = jnp.sqrt(r2)
        sh = (np.sqrt(3.0).astype(np.float32) * v3) / r
        rc = jnp.maximum(r, 1e-6)
        t = 2.0 * (r / MAX_RADIUS - 1.0)
        cut = 0.5 * (1.0 - jnp.cos(np.float32(np.pi) * t))
        cut = jnp.where(t > 0.0, 0.0, cut)
        cut = jnp.where(t < -1.0, 1.0, cut)
        n = (lax.broadcasted_iota(jnp.int32, (1, NB), 1) + 1).astype(F32)
        feat = (np.float32(np.sqrt(2.0 / MAX_RADIUS) * np.sqrt(NB))
                * jnp.sin(n * (np.float32(np.pi) / MAX_RADIUS) * rc) / rc)
        h = _silu(_dot(feat, wfc1_r[...]) + bfc1_r[...])
        we = _dot(h, wfc2_r[...])
        shf = _dot(cut * sh, wshw_r[...])
        out_r[...] = we * shf

    return pl.pallas_call(
        body,
        grid=(E // _BE,),
        in_specs=[
            _rows(8, _BE), _rows(8, _BE), _full2d((NB, RH)), _full2d((1, RH)),
            _full2d((RH, D_CONV)), _full2d((3, D_CONV)),
        ],
        out_specs=_rows(D_CONV, _BE),
        out_shape=jax.ShapeDtypeStruct((E, D_CONV), F32),
    )(vs, vd, wfc1, bfc1, wfc2, wsh_w)


def _gate_update(y, a0, a1, sc, wl2):
    agg = (a0 + a1) * np.float32(1.0 / np.sqrt(NUM_NEIGHBORS))
    z = _dot(agg, wl2) + sc
    scal = _silu(z[:, :NS])
    gates = _sigmoid(z[:, NS:NS + NG])
    ri = lax.broadcasted_iota(jnp.int32, (NG, NV * 3), 0)
    ci = lax.broadcasted_iota(jnp.int32, (NG, NV * 3), 1)
    sel = (ci // 3 == ri).astype(F32)
    vecp = z[:, NS + NG:] * _dot(gates, sel)
    p1r = lax.broadcasted_iota(jnp.int32, (NS, D_H), 0)
    p1c = lax.broadcasted_iota(jnp.int32, (NS, D_H), 1)
    p1 = (p1r == p1c).astype(F32)
    p2r = lax.broadcasted_iota(jnp.int32, (NV * 3, D_H), 0)
    p2c = lax.broadcasted_iota(jnp.int32, (NV * 3, D_H), 1)
    p2 = (p2r + NS == p2c).astype(F32)
    out = _dot(scal, p1) + _dot(vecp, p2)
    return y + np.float32(H_STEP) * out


def _tc_mid(y, na, a0, a1, sc, wl2, mp, wna, wl1, wmsg, wsc):
    def body(y_r, na_r, a0_r, a1_r, sc_r, wl2_r, mp_r, wna_r, wl1_r, wmsg_r, wsc_r,
             yn_r, y3_r, scn_r, xpad_r):
        yn = _gate_update(y_r[...], a0_r[...], a1_r[...], sc_r[...], wl2_r[...])
        y3, scn = _node_front(yn, na_r[...], wna_r[...], wl1_r[...], wmsg_r[...], wsc_r[...])
        yn_r[...] = yn
        y3_r[...] = y3
        scn_r[...] = scn
        xpad_r[...] = _dot(yn, mp_r[...])

    return pl.pallas_call(
        body,
        grid=(N // _BN,),
        in_specs=[
            _rows(D_H), _rows(32), _rows(D_CONV), _rows(D_CONV), _rows(D_CONV),
            _full2d((D_CONV, D_CONV)), _full2d((D_H, 8)), _full2d((32, D_H)),
            _full2d((D_H, D_H)), _full2d((D_H, D_CONV)), _full2d((D_H, D_CONV)),
        ],
        out_specs=[_rows(D_H), _rows(D_CONV), _rows(D_CONV), _rows(8)],
        out_shape=[
            jax.ShapeDtypeStruct((N, D_H), F32),
            jax.ShapeDtypeStruct((N, D_CONV), F32),
            jax.ShapeDtypeStruct((N, D_CONV), F32),
            jax.ShapeDtypeStruct((N, 8), F32),
        ],
    )(y, na, a0, a1, sc, wl2, mp, wna, wl1, wmsg, wsc)


def _tc_fin(y, a0, a1, sc, wl2, m):
    def body(y_r, a0_r, a1_r, sc_r, wl2_r, m_r, x_r):
        yn = _gate_update(y_r[...], a0_r[...], a1_r[...], sc_r[...], wl2_r[...])
        x_r[...] = _dot(yn, m_r[...])

    return pl.pallas_call(
        body,
        grid=(N // _BN,),
        in_specs=[
            _rows(D_H), _rows(D_CONV), _rows(D_CONV), _rows(D_CONV),
            _full2d((D_CONV, D_CONV)), _full2d((D_H, D_IN)),
        ],
        out_specs=_rows(D_IN),
        out_shape=jax.ShapeDtypeStruct((N, D_IN), F32),
    )(y, a0, a1, sc, wl2, m)


def kernel(x, batch, node_attr, edge_src, edge_dst, K_PU, emb, W_na, W_lin1,
           W_fc1, b_fc1, W_fc2, W_sh, W_msg, W_lin2, W_sc):
    del batch
    G = K_PU.T @ K_PU
    w, V = jnp.linalg.eigh(G)
    G_isqrt = (V * (1.0 / jnp.sqrt(jnp.maximum(w, 1e-12)))) @ V.T
    M = K_PU @ G_isqrt

    mt = M.T
    xsel = jnp.pad(jnp.eye(3, dtype=F32), ((0, D_IN - 3), (0, 8 - 3)))
    mp = M @ xsel
    attr = node_attr.reshape(N, 1).astype(jnp.int32)
    bfc1 = b_fc1.reshape(LAYERS, 1, RH)

    y, na, y3, sc, xpad = _tc_pre(
        x, attr, emb, mt, xsel, W_na[0], W_lin1[0], W_msg[0], W_sc[0])

    for l in range(LAYERS):
        vs, vd = _sc_gather_pos(xpad, edge_src, edge_dst)
        wsh = _tc_edge(vs, vd, W_fc1[l], bfc1[l], W_fc2[l], W_sh[l])
        agg2 = _sc_scatter(y3, wsh, edge_src, edge_dst)
        a0 = agg2[0, :N]
        a1 = agg2[1, :N]
        if l + 1 < LAYERS:
            y, y3, sc, xpad = _tc_mid(y, na, a0, a1, sc, W_lin2[l], mp,
                                      W_na[l + 1], W_lin1[l + 1], W_msg[l + 1], W_sc[l + 1])
        else:
            xout = _tc_fin(y, a0, a1, sc, W_lin2[l], M)
    return xout


# speedup vs baseline: 3.6521x; 1.3672x over previous
import functools

import jax
import jax.numpy as jnp
import numpy as np
from jax import lax
from jax.experimental import pallas as pl
from jax.experimental.pallas import tpu as pltpu
from jax.experimental.pallas import tpu_sc as plsc

N = 10000
E = 640000
D_IN = 6
D_H = 80
D_CONV = 96
NS = 32
NG = 16
NV = 16
MAX_ATOM = 100
NB = 8
RH = 16
LAYERS = 2
MAX_RADIUS = 2.5
NUM_NEIGHBORS = 64.0
H_STEP = 0.1

SC_CORES = 2
SC_SUBCORES = 16
SC_WORKERS = SC_CORES * SC_SUBCORES
EPW = E // SC_WORKERS

NPAD = 10240
ROWS_PER_SUBCORE = NPAD // SC_SUBCORES

_SC_MESH = dict(core_axis_name="c", subcore_axis_name="s")

F32 = jnp.float32


def _silu(v):
    return v / (1.0 + jnp.exp(-v))


def _sigmoid(v):
    return 1.0 / (1.0 + jnp.exp(-v))


_CP = 4000


def _sc_gather_pos(xpad, edge_src, edge_dst):
    mesh = plsc.VectorSubcoreMesh(**_SC_MESH)

    @functools.partial(
        pl.kernel,
        out_type=(
            jax.ShapeDtypeStruct((E, 8), F32),
            jax.ShapeDtypeStruct((E, 8), F32),
        ),
        mesh=mesh,
        scratch_types=[
            pltpu.VMEM((_CP,), jnp.int32),
            pltpu.VMEM((_CP,), jnp.int32),
            pltpu.VMEM((_CP, 8), F32),
            pltpu.VMEM((_CP, 8), F32),
            pltpu.SemaphoreType.DMA,
            pltpu.SemaphoreType.DMA,
        ],
        compiler_params=pltpu.CompilerParams(use_tc_tiling_on_sc=False),
    )
    def k(xpad_hbm, src_hbm, dst_hbm, outs_hbm, outd_hbm, sidx, didx, sbuf, dbuf, sem1, sem2):
        wid = lax.axis_index("s") * SC_CORES + lax.axis_index("c")
        base = wid * EPW

        def chunk(i, _):
            eb = base + i * _CP
            pltpu.sync_copy(src_hbm.at[pl.ds(eb, _CP)], sidx)
            pltpu.sync_copy(dst_hbm.at[pl.ds(eb, _CP)], didx)
            c1 = pltpu.async_copy(xpad_hbm.at[sidx], sbuf, sem1)
            c2 = pltpu.async_copy(xpad_hbm.at[didx], dbuf, sem2)
            c1.wait()
            c2.wait()
            pltpu.sync_copy(sbuf, outs_hbm.at[pl.ds(eb, _CP)])
            pltpu.sync_copy(dbuf, outd_hbm.at[pl.ds(eb, _CP)])
            return 0

        lax.fori_loop(0, EPW // _CP, chunk, 0)

    return k(xpad, edge_src, edge_dst)


_CS = 200


def _sc_scatter(y3, wsh, edge_src, edge_dst):
    mesh = plsc.VectorSubcoreMesh(**_SC_MESH)

    @functools.partial(
        pl.kernel,
        out_type=jax.ShapeDtypeStruct((SC_CORES, NPAD, D_CONV), F32),
        mesh=mesh,
        scratch_types=[
            pltpu.VMEM((_CS,), jnp.int32),
            pltpu.VMEM((_CS,), jnp.int32),
            pltpu.VMEM((_CS, D_CONV), F32),
            pltpu.VMEM((_CS, D_CONV), F32),
            pltpu.VMEM_SHARED((NPAD, D_CONV), F32),
            pltpu.SemaphoreType.DMA,
        ],
        compiler_params=pltpu.CompilerParams(use_tc_tiling_on_sc=False),
    )
    def k(y3_hbm, wsh_hbm, src_hbm, dst_hbm, out_hbm, sidx, didx, y3b, wshb, aggsh, sem):
        cid = lax.axis_index("c")
        sid = lax.axis_index("s")
        wid = sid * SC_CORES + cid
        base = wid * EPW
        r0 = sid * ROWS_PER_SUBCORE

        def zrow(i, _):
            for kk in range(D_CONV // 16):
                wshb[i, pl.ds(kk * 16, 16)] = jnp.zeros((16,), F32)
            return 0

        lax.fori_loop(0, _CS, zrow, 0)
        for off in range(0, ROWS_PER_SUBCORE, _CS):
            m = min(_CS, ROWS_PER_SUBCORE - off)
            pltpu.sync_copy(wshb.at[pl.ds(0, m)], aggsh.at[pl.ds(r0 + off, m)])
        plsc.subcore_barrier()

        def chunk(i, _):
            eb = base + i * _CS
            pltpu.sync_copy(src_hbm.at[pl.ds(eb, _CS)], sidx)
            pltpu.sync_copy(dst_hbm.at[pl.ds(eb, _CS)], didx)
            g = pltpu.async_copy(y3_hbm.at[sidx], y3b, sem)
            pltpu.sync_copy(wsh_hbm.at[pl.ds(eb, _CS)], wshb)
            g.wait()

            def mrow(r, __):
                for kk in range(D_CONV // 16):
                    sl = pl.ds(kk * 16, 16)
                    wshb[r, sl] = wshb[r, sl] * y3b[r, sl]
                return 0

            lax.fori_loop(0, _CS, mrow, 0)
            pltpu.sync_copy(wshb, aggsh.at[didx], add=True)
            return 0

        lax.fori_loop(0, EPW // _CS, chunk, 0)
        plsc.subcore_barrier()

        for off in range(0, ROWS_PER_SUBCORE, _CS):
            m = min(_CS, ROWS_PER_SUBCORE - off)
            pltpu.sync_copy(aggsh.at[pl.ds(r0 + off, m)], y3b.at[pl.ds(0, m)])
            pltpu.sync_copy(y3b.at[pl.ds(0, m)], out_hbm.at[cid, pl.ds(r0 + off, m)])

    return k(y3, wsh, edge_src, edge_dst)


_BN = 2000
_BE = 4000


def _dot(a, b):
    return jnp.dot(a, b, preferred_element_type=F32)


def _full2d(shape):
    return pl.BlockSpec(shape, lambda i: (0, 0))


def _rows(width, block=_BN):
    return pl.BlockSpec((block, width), lambda i: (i, 0))


def _node_front(y, na, wna, wl1, wmsg, wsc):
    s = _silu(_dot(na, wna))
    ys = y * s
    y3 = _dot(_dot(ys, wl1), wmsg)
    sc = _dot(ys, wsc)
    return y3, sc


def _tc_pre(x, attr, emb, mt, mp, wna, wl1, wmsg, wsc):
    def body(x_r, attr_r, emb_r, mt_r, mp_r, wna_r, wl1_r, wmsg_r, wsc_r,
             y_r, na_r, y3_r, sc_r, xpad_r):
        xb = x_r[...]
        onehot = (attr_r[...] == lax.broadcasted_iota(jnp.int32, (1, MAX_ATOM), 1)).astype(F32)
        na = _dot(onehot, emb_r[...])
        y = _dot(xb, mt_r[...])
        y3, sc = _node_front(y, na, wna_r[...], wl1_r[...], wmsg_r[...], wsc_r[...])
        y_r[...] = y
        na_r[...] = na
        y3_r[...] = y3
        sc_r[...] = sc
        xpad_r[...] = _dot(xb, mp_r[...])

    return pl.pallas_call(
        body,
        grid=(N // _BN,),
        in_specs=[
            _rows(D_IN), _rows(1), _full2d((MAX_ATOM, 32)), _full2d((D_IN, D_H)),
            _full2d((D_IN, 8)), _full2d((32, D_H)), _full2d((D_H, D_H)),
            _full2d((D_H, D_CONV)), _full2d((D_H, D_CONV)),
        ],
        out_specs=[_rows(D_H), _rows(32), _rows(D_CONV), _rows(D_CONV), _rows(8)],
        out_shape=[
            jax.ShapeDtypeStruct((N, D_H), F32),
            jax.ShapeDtypeStruct((N, 32), F32),
            jax.ShapeDtypeStruct((N, D_CONV), F32),
            jax.ShapeDtypeStruct((N, D_CONV), F32),
            jax.ShapeDtypeStruct((N, 8), F32),
        ],
    )(x, attr, emb, mt, mp, wna, wl1, wmsg, wsc)


def _tc_edge(vs, vd, wfc1, bfc1, wfc2, wsh_w):
    def body(vs_r, vd_r, wfc1_r, bfc1_r, wfc2_r, wshw_r, out_r):
        v3 = vs_r[...][:, :3] - vd_r[...][:, :3]
        r2 = jnp.sum(v3 * v3, axis=1, keepdims=True) + 1e-12
        r = jnp.sqrt(r2)
        sh = (np.sqrt(3.0).astype(np.float32) * v3) / r
        rc = jnp.maximum(r, 1e-6)
        n = (lax.broadcasted_iota(jnp.int32, (1, NB), 1) + 1).astype(F32)
        sines = jnp.sin(n * (np.float32(np.pi) / MAX_RADIUS) * rc)
        feat = (np.float32(np.sqrt(2.0 / MAX_RADIUS) * np.sqrt(NB)) * sines / rc)
        s1 = sines[:, 0:1]
        cut = s1 * s1
        cut = jnp.where(r > MAX_RADIUS, 0.0, cut)
        cut = jnp.where(r < 0.5 * MAX_RADIUS, 1.0, cut)
        h = _silu(_dot(feat, wfc1_r[...]) + bfc1_r[...])
        we = _dot(h, wfc2_r[...])
        shf = _dot(cut * sh, wshw_r[...])
        out_r[...] = we * shf

    return pl.pallas_call(
        body,
        grid=(E // _BE,),
        in_specs=[
            _rows(8, _BE), _rows(8, _BE), _full2d((NB, RH)), _full2d((1, RH)),
            _full2d((RH, D_CONV)), _full2d((3, D_CONV)),
        ],
        out_specs=_rows(D_CONV, _BE),
        out_shape=jax.ShapeDtypeStruct((E, D_CONV), F32),
    )(vs, vd, wfc1, bfc1, wfc2, wsh_w)


def _gate_update(y, a0, a1, sc, wl2):
    agg = (a0 + a1) * np.float32(1.0 / np.sqrt(NUM_NEIGHBORS))
    z = _dot(agg, wl2) + sc
    scal = _silu(z[:, :NS])
    gates = _sigmoid(z[:, NS:NS + NG])
    ri = lax.broadcasted_iota(jnp.int32, (NG, NV * 3), 0)
    ci = lax.broadcasted_iota(jnp.int32, (NG, NV * 3), 1)
    sel = (ci // 3 == ri).astype(F32)
    vecp = z[:, NS + NG:] * _dot(gates, sel)
    p1r = lax.broadcasted_iota(jnp.int32, (NS, D_H), 0)
    p1c = lax.broadcasted_iota(jnp.int32, (NS, D_H), 1)
    p1 = (p1r == p1c).astype(F32)
    p2r = lax.broadcasted_iota(jnp.int32, (NV * 3, D_H), 0)
    p2c = lax.broadcasted_iota(jnp.int32, (NV * 3, D_H), 1)
    p2 = (p2r + NS == p2c).astype(F32)
    out = _dot(scal, p1) + _dot(vecp, p2)
    return y + np.float32(H_STEP) * out


def _tc_mid(y, na, a0, a1, sc, wl2, mp, wna, wl1, wmsg, wsc):
    def body(y_r, na_r, a0_r, a1_r, sc_r, wl2_r, mp_r, wna_r, wl1_r, wmsg_r, wsc_r,
             yn_r, y3_r, scn_r, xpad_r):
        yn = _gate_update(y_r[...], a0_r[...], a1_r[...], sc_r[...], wl2_r[...])
        y3, scn = _node_front(yn, na_r[...], wna_r[...], wl1_r[...], wmsg_r[...], wsc_r[...])
        yn_r[...] = yn
        y3_r[...] = y3
        scn_r[...] = scn
        xpad_r[...] = _dot(yn, mp_r[...])

    return pl.pallas_call(
        body,
        grid=(N // _BN,),
        in_specs=[
            _rows(D_H), _rows(32), _rows(D_CONV), _rows(D_CONV), _rows(D_CONV),
            _full2d((D_CONV, D_CONV)), _full2d((D_H, 8)), _full2d((32, D_H)),
            _full2d((D_H, D_H)), _full2d((D_H, D_CONV)), _full2d((D_H, D_CONV)),
        ],
        out_specs=[_rows(D_H), _rows(D_CONV), _rows(D_CONV), _rows(8)],
        out_shape=[
            jax.ShapeDtypeStruct((N, D_H), F32),
            jax.ShapeDtypeStruct((N, D_CONV), F32),
            jax.ShapeDtypeStruct((N, D_CONV), F32),
            jax.ShapeDtypeStruct((N, 8), F32),
        ],
    )(y, na, a0, a1, sc, wl2, mp, wna, wl1, wmsg, wsc)


def _tc_fin(y, a0, a1, sc, wl2, m):
    def body(y_r, a0_r, a1_r, sc_r, wl2_r, m_r, x_r):
        yn = _gate_update(y_r[...], a0_r[...], a1_r[...], sc_r[...], wl2_r[...])
        x_r[...] = _dot(yn, m_r[...])

    return pl.pallas_call(
        body,
        grid=(N // _BN,),
        in_specs=[
            _rows(D_H), _rows(D_CONV), _rows(D_CONV), _rows(D_CONV),
            _full2d((D_CONV, D_CONV)), _full2d((D_H, D_IN)),
        ],
        out_specs=_rows(D_IN),
        out_shape=jax.ShapeDtypeStruct((N, D_IN), F32),
    )(y, a0, a1, sc, wl2, m)


def kernel(x, batch, node_attr, edge_src, edge_dst, K_PU, emb, W_na, W_lin1,
           W_fc1, b_fc1, W_fc2, W_sh, W_msg, W_lin2, W_sc):
    del batch
    G = K_PU.T @ K_PU
    w, V = jnp.linalg.eigh(G)
    G_isqrt = (V * (1.0 / jnp.sqrt(jnp.maximum(w, 1e-12)))) @ V.T
    M = K_PU @ G_isqrt

    mt = M.T
    xsel = jnp.pad(jnp.eye(3, dtype=F32), ((0, D_IN - 3), (0, 8 - 3)))
    mp = M @ xsel
    attr = node_attr.reshape(N, 1).astype(jnp.int32)
    bfc1 = b_fc1.reshape(LAYERS, 1, RH)

    y, na, y3, sc, xpad = _tc_pre(
        x, attr, emb, mt, xsel, W_na[0], W_lin1[0], W_msg[0], W_sc[0])

    for l in range(LAYERS):
        vs, vd = _sc_gather_pos(xpad, edge_src, edge_dst)
        wsh = _tc_edge(vs, vd, W_fc1[l], bfc1[l], W_fc2[l], W_sh[l])
        agg2 = _sc_scatter(y3, wsh, edge_src, edge_dst)
        a0 = agg2[0, :N]
        a1 = agg2[1, :N]
        if l + 1 < LAYERS:
            y, y3, sc, xpad = _tc_mid(y, na, a0, a1, sc, W_lin2[l], mp,
                                      W_na[l + 1], W_lin1[l + 1], W_msg[l + 1], W_sc[l + 1])
        else:
            xout = _tc_fin(y, a0, a1, sc, W_lin2[l], M)
    return xout


# speedup vs baseline: 5.3691x; 1.4701x over previous
import functools

import jax
import jax.numpy as jnp
import numpy as np
from jax import lax
from jax.experimental import pallas as pl
from jax.experimental.pallas import tpu as pltpu
from jax.experimental.pallas import tpu_sc as plsc

N = 10000
E = 640000
D_IN = 6
D_H = 80
D_CONV = 96
NS = 32
NG = 16
NV = 16
MAX_ATOM = 100
NB = 8
RH = 16
LAYERS = 2
MAX_RADIUS = 2.5
NUM_NEIGHBORS = 64.0
H_STEP = 0.1

SC_CORES = 2
SC_SUBCORES = 16
SC_WORKERS = SC_CORES * SC_SUBCORES
EPW = E // SC_WORKERS

NPAD = 10240
ROWS_PER_SUBCORE = NPAD // SC_SUBCORES

_SC_MESH = dict(core_axis_name="c", subcore_axis_name="s")

F32 = jnp.float32


def _silu(v):
    return v / (1.0 + jnp.exp(-v))


def _sigmoid(v):
    return 1.0 / (1.0 + jnp.exp(-v))


_CP = 4000


def _sc_gather_pos(xpad, edge_src, edge_dst):
    mesh = plsc.VectorSubcoreMesh(**_SC_MESH)

    @functools.partial(
        pl.kernel,
        out_type=(
            jax.ShapeDtypeStruct((E, 8), F32),
            jax.ShapeDtypeStruct((E, 8), F32),
        ),
        mesh=mesh,
        scratch_types=[
            pltpu.VMEM((_CP,), jnp.int32),
            pltpu.VMEM((_CP,), jnp.int32),
            pltpu.VMEM((_CP, 8), F32),
            pltpu.VMEM((_CP, 8), F32),
            pltpu.SemaphoreType.DMA,
            pltpu.SemaphoreType.DMA,
        ],
        compiler_params=pltpu.CompilerParams(use_tc_tiling_on_sc=False),
    )
    def k(xpad_hbm, src_hbm, dst_hbm, outs_hbm, outd_hbm, sidx, didx, sbuf, dbuf, sem1, sem2):
        wid = lax.axis_index("s") * SC_CORES + lax.axis_index("c")
        base = wid * EPW

        def chunk(i, _):
            eb = base + i * _CP
            pltpu.sync_copy(src_hbm.at[pl.ds(eb, _CP)], sidx)
            pltpu.sync_copy(dst_hbm.at[pl.ds(eb, _CP)], didx)
            c1 = pltpu.async_copy(xpad_hbm.at[sidx], sbuf, sem1)
            c2 = pltpu.async_copy(xpad_hbm.at[didx], dbuf, sem2)
            c1.wait()
            c2.wait()
            pltpu.sync_copy(sbuf, outs_hbm.at[pl.ds(eb, _CP)])
            pltpu.sync_copy(dbuf, outd_hbm.at[pl.ds(eb, _CP)])
            return 0

        lax.fori_loop(0, EPW // _CP, chunk, 0)

    return k(xpad, edge_src, edge_dst)


_CS = 200


def _sc_scatter(y3, wsh, edge_src, edge_dst):
    mesh = plsc.VectorSubcoreMesh(**_SC_MESH)

    @functools.partial(
        pl.kernel,
        out_type=jax.ShapeDtypeStruct((SC_CORES, NPAD, D_CONV), F32),
        mesh=mesh,
        scratch_types=[
            pltpu.VMEM((_CS,), jnp.int32),
            pltpu.VMEM((_CS,), jnp.int32),
            pltpu.VMEM((_CS, D_CONV), F32),
            pltpu.VMEM((_CS, D_CONV), F32),
            pltpu.VMEM_SHARED((NPAD, D_CONV), F32),
            pltpu.SemaphoreType.DMA,
        ],
        compiler_params=pltpu.CompilerParams(use_tc_tiling_on_sc=False),
    )
    def k(y3_hbm, wsh_hbm, src_hbm, dst_hbm, out_hbm, sidx, didx, y3b, wshb, aggsh, sem):
        cid = lax.axis_index("c")
        sid = lax.axis_index("s")
        wid = sid * SC_CORES + cid
        base = wid * EPW
        r0 = sid * ROWS_PER_SUBCORE

        def zrow(i, _):
            for kk in range(D_CONV // 16):
                wshb[i, pl.ds(kk * 16, 16)] = jnp.zeros((16,), F32)
            return 0

        lax.fori_loop(0, _CS, zrow, 0)
        for off in range(0, ROWS_PER_SUBCORE, _CS):
            m = min(_CS, ROWS_PER_SUBCORE - off)
            pltpu.sync_copy(wshb.at[pl.ds(0, m)], aggsh.at[pl.ds(r0 + off, m)])
        plsc.subcore_barrier()

        def chunk(i, _):
            eb = base + i * _CS
            pltpu.sync_copy(src_hbm.at[pl.ds(eb, _CS)], sidx)
            pltpu.sync_copy(dst_hbm.at[pl.ds(eb, _CS)], didx)
            g = pltpu.async_copy(y3_hbm.at[sidx], y3b, sem)
            pltpu.sync_copy(wsh_hbm.at[pl.ds(eb, _CS)], wshb)
            g.wait()

            def mrow(r, __):
                for kk in range(D_CONV // 16):
                    sl = pl.ds(kk * 16, 16)
                    wshb[r, sl] = wshb[r, sl] * y3b[r, sl]
                return 0

            lax.fori_loop(0, _CS, mrow, 0)
            pltpu.sync_copy(wshb, aggsh.at[didx], add=True)
            return 0

        lax.fori_loop(0, EPW // _CS, chunk, 0)
        plsc.subcore_barrier()

        for off in range(0, ROWS_PER_SUBCORE, _CS):
            m = min(_CS, ROWS_PER_SUBCORE - off)
            pltpu.sync_copy(aggsh.at[pl.ds(r0 + off, m)], y3b.at[pl.ds(0, m)])
            pltpu.sync_copy(y3b.at[pl.ds(0, m)], out_hbm.at[cid, pl.ds(r0 + off, m)])

    return k(y3, wsh, edge_src, edge_dst)


_BN = 2000
_BE = 4000


def _dot(a, b):
    return jnp.dot(a, b, preferred_element_type=F32)


def _full2d(shape):
    return pl.BlockSpec(shape, lambda i: (0, 0))


def _rows(width, block=_BN):
    return pl.BlockSpec((block, width), lambda i: (i, 0))


def _node_front(y, na, wna, wl1, wmsg, wsc):
    s = _silu(_dot(na, wna))
    ys = y * s
    y3 = _dot(_dot(ys, wl1), wmsg)
    sc = _dot(ys, wsc)
    return y3, sc


def _tc_pre(x, attr, emb, mt, mp, wna, wl1, wmsg, wsc):
    def body(x_r, attr_r, emb_r, mt_r, mp_r, wna_r, wl1_r, wmsg_r, wsc_r,
             y_r, na_r, y3_r, sc_r, xpad_r):
        xb = x_r[...]
        onehot = (attr_r[...] == lax.broadcasted_iota(jnp.int32, (1, MAX_ATOM), 1)).astype(F32)
        na = _dot(onehot, emb_r[...])
        y = _dot(xb, mt_r[...])
        y3, sc = _node_front(y, na, wna_r[...], wl1_r[...], wmsg_r[...], wsc_r[...])
        y_r[...] = y
        na_r[...] = na
        y3_r[...] = y3
        sc_r[...] = sc
        xpad_r[...] = _dot(xb, mp_r[...])

    return pl.pallas_call(
        body,
        grid=(N // _BN,),
        in_specs=[
            _rows(D_IN), _rows(1), _full2d((MAX_ATOM, 32)), _full2d((D_IN, D_H)),
            _full2d((D_IN, 8)), _full2d((32, D_H)), _full2d((D_H, D_H)),
            _full2d((D_H, D_CONV)), _full2d((D_H, D_CONV)),
        ],
        out_specs=[_rows(D_H), _rows(32), _rows(D_CONV), _rows(D_CONV), _rows(8)],
        out_shape=[
            jax.ShapeDtypeStruct((N, D_H), F32),
            jax.ShapeDtypeStruct((N, 32), F32),
            jax.ShapeDtypeStruct((N, D_CONV), F32),
            jax.ShapeDtypeStruct((N, D_CONV), F32),
            jax.ShapeDtypeStruct((N, 8), F32),
        ],
    )(x, attr, emb, mt, mp, wna, wl1, wmsg, wsc)


def _tc_edge(vs, vd, wfc1, bfc1, wfc2, wsh_w):
    def body(vs_r, vd_r, wfc1_r, bfc1_r, wfc2_r, wshw_r, out_r):
        v3 = vs_r[...][:, :3] - vd_r[...][:, :3]
        r2 = jnp.sum(v3 * v3, axis=1, keepdims=True) + 1e-12
        r = jnp.sqrt(r2)
        sh = (np.sqrt(3.0).astype(np.float32) * v3) / r
        ones3 = jnp.ones((1, 3), dtype=F32)
        r2t = lax.dot_general(ones3, v3 * v3, (((1,), (1,)), ((), ())),
                              preferred_element_type=F32) + 1e-12
        rt = jnp.maximum(jnp.sqrt(r2t), 1e-6)
        th = (np.float32(np.pi) / MAX_RADIUS) * rt
        s1 = jnp.sin(th)
        c2 = 2.0 * jnp.cos(th)
        sins = [s1, c2 * s1]
        for _ in range(NB - 2):
            sins.append(c2 * sins[-1] - sins[-2])
        feat_t = (np.float32(np.sqrt(2.0 / MAX_RADIUS) * np.sqrt(NB))
                  * jnp.concatenate(sins, axis=0) / rt)
        eye8 = (lax.broadcasted_iota(jnp.int32, (NB, NB), 0)
                == lax.broadcasted_iota(jnp.int32, (NB, NB), 1)).astype(F32)
        feat = lax.dot_general(feat_t, eye8, (((0,), (0,)), ((), ())),
                               preferred_element_type=F32)
        cutt = s1 * s1
        cutt = jnp.where(rt > MAX_RADIUS, 0.0, cutt)
        cutt = jnp.where(rt < 0.5 * MAX_RADIUS, 1.0, cutt)
        ones1 = jnp.ones((1, 1), dtype=F32)
        cut = lax.dot_general(cutt, ones1, (((0,), (0,)), ((), ())),
                              preferred_element_type=F32)
        h = _silu(_dot(feat, wfc1_r[...]) + bfc1_r[...])
        we = _dot(h, wfc2_r[...])
        shf = _dot(cut * sh, wshw_r[...])
        out_r[...] = we * shf

    return pl.pallas_call(
        body,
        grid=(E // _BE,),
        in_specs=[
            _rows(8, _BE), _rows(8, _BE), _full2d((NB, RH)), _full2d((1, RH)),
            _full2d((RH, D_CONV)), _full2d((3, D_CONV)),
        ],
        out_specs=_rows(D_CONV, _BE),
        out_shape=jax.ShapeDtypeStruct((E, D_CONV), F32),
    )(vs, vd, wfc1, bfc1, wfc2, wsh_w)


def _gate_update(y, a0, a1, sc, wl2):
    agg = (a0 + a1) * np.float32(1.0 / np.sqrt(NUM_NEIGHBORS))
    z = _dot(agg, wl2) + sc
    scal = _silu(z[:, :NS])
    gates = _sigmoid(z[:, NS:NS + NG])
    ri = lax.broadcasted_iota(jnp.int32, (NG, NV * 3), 0)
    ci = lax.broadcasted_iota(jnp.int32, (NG, NV * 3), 1)
    sel = (ci // 3 == ri).astype(F32)
    vecp = z[:, NS + NG:] * _dot(gates, sel)
    p1r = lax.broadcasted_iota(jnp.int32, (NS, D_H), 0)
    p1c = lax.broadcasted_iota(jnp.int32, (NS, D_H), 1)
    p1 = (p1r == p1c).astype(F32)
    p2r = lax.broadcasted_iota(jnp.int32, (NV * 3, D_H), 0)
    p2c = lax.broadcasted_iota(jnp.int32, (NV * 3, D_H), 1)
    p2 = (p2r + NS == p2c).astype(F32)
    out = _dot(scal, p1) + _dot(vecp, p2)
    return y + np.float32(H_STEP) * out


def _tc_mid(y, na, a0, a1, sc, wl2, mp, wna, wl1, wmsg, wsc):
    def body(y_r, na_r, a0_r, a1_r, sc_r, wl2_r, mp_r, wna_r, wl1_r, wmsg_r, wsc_r,
             yn_r, y3_r, scn_r, xpad_r):
        yn = _gate_update(y_r[...], a0_r[...], a1_r[...], sc_r[...], wl2_r[...])
        y3, scn = _node_front(yn, na_r[...], wna_r[...], wl1_r[...], wmsg_r[...], wsc_r[...])
        yn_r[...] = yn
        y3_r[...] = y3
        scn_r[...] = scn
        xpad_r[...] = _dot(yn, mp_r[...])

    return pl.pallas_call(
        body,
        grid=(N // _BN,),
        in_specs=[
            _rows(D_H), _rows(32), _rows(D_CONV), _rows(D_CONV), _rows(D_CONV),
            _full2d((D_CONV, D_CONV)), _full2d((D_H, 8)), _full2d((32, D_H)),
            _full2d((D_H, D_H)), _full2d((D_H, D_CONV)), _full2d((D_H, D_CONV)),
        ],
        out_specs=[_rows(D_H), _rows(D_CONV), _rows(D_CONV), _rows(8)],
        out_shape=[
            jax.ShapeDtypeStruct((N, D_H), F32),
            jax.ShapeDtypeStruct((N, D_CONV), F32),
            jax.ShapeDtypeStruct((N, D_CONV), F32),
            jax.ShapeDtypeStruct((N, 8), F32),
        ],
    )(y, na, a0, a1, sc, wl2, mp, wna, wl1, wmsg, wsc)


def _tc_fin(y, a0, a1, sc, wl2, m):
    def body(y_r, a0_r, a1_r, sc_r, wl2_r, m_r, x_r):
        yn = _gate_update(y_r[...], a0_r[...], a1_r[...], sc_r[...], wl2_r[...])
        x_r[...] = _dot(yn, m_r[...])

    return pl.pallas_call(
        body,
        grid=(N // _BN,),
        in_specs=[
            _rows(D_H), _rows(D_CONV), _rows(D_CONV), _rows(D_CONV),
            _full2d((D_CONV, D_CONV)), _full2d((D_H, D_IN)),
        ],
        out_specs=_rows(D_IN),
        out_shape=jax.ShapeDtypeStruct((N, D_IN), F32),
    )(y, a0, a1, sc, wl2, m)


def kernel(x, batch, node_attr, edge_src, edge_dst, K_PU, emb, W_na, W_lin1,
           W_fc1, b_fc1, W_fc2, W_sh, W_msg, W_lin2, W_sc):
    del batch
    G = K_PU.T @ K_PU
    w, V = jnp.linalg.eigh(G)
    G_isqrt = (V * (1.0 / jnp.sqrt(jnp.maximum(w, 1e-12)))) @ V.T
    M = K_PU @ G_isqrt

    mt = M.T
    xsel = jnp.pad(jnp.eye(3, dtype=F32), ((0, D_IN - 3), (0, 8 - 3)))
    mp = M @ xsel
    attr = node_attr.reshape(N, 1).astype(jnp.int32)
    bfc1 = b_fc1.reshape(LAYERS, 1, RH)

    y, na, y3, sc, xpad = _tc_pre(
        x, attr, emb, mt, xsel, W_na[0], W_lin1[0], W_msg[0], W_sc[0])

    for l in range(LAYERS):
        vs, vd = _sc_gather_pos(xpad, edge_src, edge_dst)
        wsh = _tc_edge(vs, vd, W_fc1[l], bfc1[l], W_fc2[l], W_sh[l])
        agg2 = _sc_scatter(y3, wsh, edge_src, edge_dst)
        a0 = agg2[0, :N]
        a1 = agg2[1, :N]
        if l + 1 < LAYERS:
            y, y3, sc, xpad = _tc_mid(y, na, a0, a1, sc, W_lin2[l], mp,
                                      W_na[l + 1], W_lin1[l + 1], W_msg[l + 1], W_sc[l + 1])
        else:
            xout = _tc_fin(y, a0, a1, sc, W_lin2[l], M)
    return xout


# speedup vs baseline: 5.4807x; 1.0208x over previous
import functools

import jax
import jax.numpy as jnp
import numpy as np
from jax import lax
from jax.experimental import pallas as pl
from jax.experimental.pallas import tpu as pltpu
from jax.experimental.pallas import tpu_sc as plsc

N = 10000
E = 640000
D_IN = 6
D_H = 80
D_CONV = 96
NS = 32
NG = 16
NV = 16
MAX_ATOM = 100
NB = 8
RH = 16
LAYERS = 2
MAX_RADIUS = 2.5
NUM_NEIGHBORS = 64.0
H_STEP = 0.1

SC_CORES = 2
SC_SUBCORES = 16
SC_WORKERS = SC_CORES * SC_SUBCORES
EPW = E // SC_WORKERS

NPAD = 10240
ROWS_PER_SUBCORE = NPAD // SC_SUBCORES

_SC_MESH = dict(core_axis_name="c", subcore_axis_name="s")

F32 = jnp.float32


def _silu(v):
    return v / (1.0 + jnp.exp(-v))


def _sigmoid(v):
    return 1.0 / (1.0 + jnp.exp(-v))


_CP = 4000


def _sc_gather_pos(xpad, edge_src, edge_dst):
    mesh = plsc.VectorSubcoreMesh(**_SC_MESH)

    @functools.partial(
        pl.kernel,
        out_type=(
            jax.ShapeDtypeStruct((E, 8), F32),
            jax.ShapeDtypeStruct((E, 8), F32),
        ),
        mesh=mesh,
        scratch_types=[
            pltpu.VMEM((_CP,), jnp.int32),
            pltpu.VMEM((_CP,), jnp.int32),
            pltpu.VMEM((_CP, 8), F32),
            pltpu.VMEM((_CP, 8), F32),
            pltpu.SemaphoreType.DMA,
            pltpu.SemaphoreType.DMA,
        ],
        compiler_params=pltpu.CompilerParams(use_tc_tiling_on_sc=False),
    )
    def k(xpad_hbm, src_hbm, dst_hbm, outs_hbm, outd_hbm, sidx, didx, sbuf, dbuf, sem1, sem2):
        wid = lax.axis_index("s") * SC_CORES + lax.axis_index("c")
        base = wid * EPW

        def chunk(i, _):
            eb = base + i * _CP
            pltpu.sync_copy(src_hbm.at[pl.ds(eb, _CP)], sidx)
            pltpu.sync_copy(dst_hbm.at[pl.ds(eb, _CP)], didx)
            c1 = pltpu.async_copy(xpad_hbm.at[sidx], sbuf, sem1)
            c2 = pltpu.async_copy(xpad_hbm.at[didx], dbuf, sem2)
            c1.wait()
            c2.wait()
            pltpu.sync_copy(sbuf, outs_hbm.at[pl.ds(eb, _CP)])
            pltpu.sync_copy(dbuf, outd_hbm.at[pl.ds(eb, _CP)])
            return 0

        lax.fori_loop(0, EPW // _CP, chunk, 0)

    return k(xpad, edge_src, edge_dst)


_CS = 80


def _sc_scatter(y3, wsh, edge_src, edge_dst):
    mesh = plsc.VectorSubcoreMesh(**_SC_MESH)

    @functools.partial(
        pl.kernel,
        out_type=jax.ShapeDtypeStruct((SC_CORES, NPAD, D_CONV), F32),
        mesh=mesh,
        scratch_types=[
            pltpu.VMEM((_CS,), jnp.int32),
            pltpu.VMEM((_CS,), jnp.int32),
            pltpu.VMEM((_CS,), jnp.int32),
            pltpu.VMEM((_CS,), jnp.int32),
            pltpu.VMEM((_CS, D_CONV), F32),
            pltpu.VMEM((_CS, D_CONV), F32),
            pltpu.VMEM((_CS, D_CONV), F32),
            pltpu.VMEM((_CS, D_CONV), F32),
            pltpu.VMEM_SHARED((NPAD, D_CONV), F32),
            pltpu.SemaphoreType.DMA,
            pltpu.SemaphoreType.DMA,
            pltpu.SemaphoreType.DMA,
            pltpu.SemaphoreType.DMA,
        ],
        compiler_params=pltpu.CompilerParams(use_tc_tiling_on_sc=False),
    )
    def k(y3_hbm, wsh_hbm, src_hbm, dst_hbm, out_hbm,
          sidx0, didx0, sidx1, didx1, y3b0, wshb0, y3b1, wshb1, aggsh,
          semg0, semw0, semg1, semw1):
        cid = lax.axis_index("c")
        sid = lax.axis_index("s")
        wid = sid * SC_CORES + cid
        base = wid * EPW
        r0 = sid * ROWS_PER_SUBCORE

        def zrow(i, _):
            for kk in range(D_CONV // 16):
                wshb0[i, pl.ds(kk * 16, 16)] = jnp.zeros((16,), F32)
            return 0

        lax.fori_loop(0, _CS, zrow, 0)
        for off in range(0, ROWS_PER_SUBCORE, _CS):
            m = min(_CS, ROWS_PER_SUBCORE - off)
            pltpu.sync_copy(wshb0.at[pl.ds(0, m)], aggsh.at[pl.ds(r0 + off, m)])
        plsc.subcore_barrier()

        def fetch(eb, sidx, didx, y3b, wshb, semg, semw):
            pltpu.sync_copy(src_hbm.at[pl.ds(eb, _CS)], sidx)
            pltpu.sync_copy(dst_hbm.at[pl.ds(eb, _CS)], didx)
            g = pltpu.async_copy(y3_hbm.at[sidx], y3b, semg)
            w = pltpu.async_copy(wsh_hbm.at[pl.ds(eb, _CS)], wshb, semw)
            return g, w

        def consume(g, w, didx, y3b, wshb):
            g.wait()
            w.wait()

            def mrow(r, __):
                for kk in range(D_CONV // 16):
                    sl = pl.ds(kk * 16, 16)
                    wshb[r, sl] = wshb[r, sl] * y3b[r, sl]
                return 0

            lax.fori_loop(0, _CS, mrow, 0)
            pltpu.sync_copy(wshb, aggsh.at[didx], add=True)

        def pair(i, _):
            eb = base + (2 * i) * _CS
            g0, w0 = fetch(eb, sidx0, didx0, y3b0, wshb0, semg0, semw0)
            g1, w1 = fetch(eb + _CS, sidx1, didx1, y3b1, wshb1, semg1, semw1)
            consume(g0, w0, didx0, y3b0, wshb0)
            consume(g1, w1, didx1, y3b1, wshb1)
            return 0

        lax.fori_loop(0, EPW // (2 * _CS), pair, 0)
        plsc.subcore_barrier()

        for off in range(0, ROWS_PER_SUBCORE, _CS):
            m = min(_CS, ROWS_PER_SUBCORE - off)
            pltpu.sync_copy(aggsh.at[pl.ds(r0 + off, m)], y3b0.at[pl.ds(0, m)])
            pltpu.sync_copy(y3b0.at[pl.ds(0, m)], out_hbm.at[cid, pl.ds(r0 + off, m)])

    return k(y3, wsh, edge_src, edge_dst)


_BN = 2000
_BE = 8000


def _dot(a, b):
    return jnp.dot(a, b, preferred_element_type=F32)


def _full2d(shape):
    return pl.BlockSpec(shape, lambda i: (0, 0))


def _rows(width, block=_BN):
    return pl.BlockSpec((block, width), lambda i: (i, 0))


def _node_front(y, na, wna, wl1, wmsg, wsc):
    s = _silu(_dot(na, wna))
    ys = y * s
    y3 = _dot(_dot(ys, wl1), wmsg)
    sc = _dot(ys, wsc)
    return y3, sc


def _tc_pre(x, attr, emb, mt, mp, wna, wl1, wmsg, wsc):
    def body(x_r, attr_r, emb_r, mt_r, mp_r, wna_r, wl1_r, wmsg_r, wsc_r,
             y_r, na_r, y3_r, sc_r, xpad_r):
        xb = x_r[...]
        onehot = (attr_r[...] == lax.broadcasted_iota(jnp.int32, (1, MAX_ATOM), 1)).astype(F32)
        na = _dot(onehot, emb_r[...])
        y = _dot(xb, mt_r[...])
        y3, sc = _node_front(y, na, wna_r[...], wl1_r[...], wmsg_r[...], wsc_r[...])
        y_r[...] = y
        na_r[...] = na
        y3_r[...] = y3
        sc_r[...] = sc
        xpad_r[...] = _dot(xb, mp_r[...])

    return pl.pallas_call(
        body,
        grid=(N // _BN,),
        in_specs=[
            _rows(D_IN), _rows(1), _full2d((MAX_ATOM, 32)), _full2d((D_IN, D_H)),
            _full2d((D_IN, 8)), _full2d((32, D_H)), _full2d((D_H, D_H)),
            _full2d((D_H, D_CONV)), _full2d((D_H, D_CONV)),
        ],
        out_specs=[_rows(D_H), _rows(32), _rows(D_CONV), _rows(D_CONV), _rows(8)],
        out_shape=[
            jax.ShapeDtypeStruct((N, D_H), F32),
            jax.ShapeDtypeStruct((N, 32), F32),
            jax.ShapeDtypeStruct((N, D_CONV), F32),
            jax.ShapeDtypeStruct((N, D_CONV), F32),
            jax.ShapeDtypeStruct((N, 8), F32),
        ],
    )(x, attr, emb, mt, mp, wna, wl1, wmsg, wsc)


def _tc_edge(vs, vd, wfc1, bfc1, wfc2, wsh_w):
    def body(vs_r, vd_r, wfc1_r, bfc1_r, wfc2_r, wshw_r, out_r):
        v3 = vs_r[...][:, :3] - vd_r[...][:, :3]
        r2 = jnp.sum(v3 * v3, axis=1, keepdims=True) + 1e-12
        r = jnp.sqrt(r2)
        sh = (np.sqrt(3.0).astype(np.float32) * v3) / r
        ones3 = jnp.ones((1, 3), dtype=F32)
        r2t = lax.dot_general(ones3, v3 * v3, (((1,), (1,)), ((), ())),
                              preferred_element_type=F32) + 1e-12
        rt = jnp.maximum(jnp.sqrt(r2t), 1e-6)
        th = (np.float32(np.pi) / MAX_RADIUS) * rt
        s1 = jnp.sin(th)
        c2 = 2.0 * jnp.cos(th)
        sins = [s1, c2 * s1]
        for _ in range(NB - 2):
            sins.append(c2 * sins[-1] - sins[-2])
        feat_t = (np.float32(np.sqrt(2.0 / MAX_RADIUS) * np.sqrt(NB))
                  * jnp.concatenate(sins, axis=0) / rt)
        eye8 = (lax.broadcasted_iota(jnp.int32, (NB, NB), 0)
                == lax.broadcasted_iota(jnp.int32, (NB, NB), 1)).astype(F32)
        feat = lax.dot_general(feat_t, eye8, (((0,), (0,)), ((), ())),
                               preferred_element_type=F32)
        cutt = s1 * s1
        cutt = jnp.where(rt > MAX_RADIUS, 0.0, cutt)
        cutt = jnp.where(rt < 0.5 * MAX_RADIUS, 1.0, cutt)
        ones1 = jnp.ones((1, 1), dtype=F32)
        cut = lax.dot_general(cutt, ones1, (((0,), (0,)), ((), ())),
                              preferred_element_type=F32)
        h = _silu(_dot(feat, wfc1_r[...]) + bfc1_r[...])
        we = _dot(h, wfc2_r[...])
        shf = _dot(cut * sh, wshw_r[...])
        out_r[...] = we * shf

    return pl.pallas_call(
        body,
        grid=(E // _BE,),
        in_specs=[
            _rows(8, _BE), _rows(8, _BE), _full2d((NB, RH)), _full2d((1, RH)),
            _full2d((RH, D_CONV)), _full2d((3, D_CONV)),
        ],
        out_specs=_rows(D_CONV, _BE),
        out_shape=jax.ShapeDtypeStruct((E, D_CONV), F32),
    )(vs, vd, wfc1, bfc1, wfc2, wsh_w)


def _gate_update(y, a0, a1, sc, wl2):
    agg = (a0 + a1) * np.float32(1.0 / np.sqrt(NUM_NEIGHBORS))
    z = _dot(agg, wl2) + sc
    scal = _silu(z[:, :NS])
    gates = _sigmoid(z[:, NS:NS + NG])
    ri = lax.broadcasted_iota(jnp.int32, (NG, NV * 3), 0)
    ci = lax.broadcasted_iota(jnp.int32, (NG, NV * 3), 1)
    sel = (ci // 3 == ri).astype(F32)
    vecp = z[:, NS + NG:] * _dot(gates, sel)
    p1r = lax.broadcasted_iota(jnp.int32, (NS, D_H), 0)
    p1c = lax.broadcasted_iota(jnp.int32, (NS, D_H), 1)
    p1 = (p1r == p1c).astype(F32)
    p2r = lax.broadcasted_iota(jnp.int32, (NV * 3, D_H), 0)
    p2c = lax.broadcasted_iota(jnp.int32, (NV * 3, D_H), 1)
    p2 = (p2r + NS == p2c).astype(F32)
    out = _dot(scal, p1) + _dot(vecp, p2)
    return y + np.float32(H_STEP) * out


def _tc_mid(y, na, a0, a1, sc, wl2, mp, wna, wl1, wmsg, wsc):
    def body(y_r, na_r, a0_r, a1_r, sc_r, wl2_r, mp_r, wna_r, wl1_r, wmsg_r, wsc_r,
             yn_r, y3_r, scn_r, xpad_r):
        yn = _gate_update(y_r[...], a0_r[...], a1_r[...], sc_r[...], wl2_r[...])
        y3, scn = _node_front(yn, na_r[...], wna_r[...], wl1_r[...], wmsg_r[...], wsc_r[...])
        yn_r[...] = yn
        y3_r[...] = y3
        scn_r[...] = scn
        xpad_r[...] = _dot(yn, mp_r[...])

    return pl.pallas_call(
        body,
        grid=(N // _BN,),
        in_specs=[
            _rows(D_H), _rows(32), _rows(D_CONV), _rows(D_CONV), _rows(D_CONV),
            _full2d((D_CONV, D_CONV)), _full2d((D_H, 8)), _full2d((32, D_H)),
            _full2d((D_H, D_H)), _full2d((D_H, D_CONV)), _full2d((D_H, D_CONV)),
        ],
        out_specs=[_rows(D_H), _rows(D_CONV), _rows(D_CONV), _rows(8)],
        out_shape=[
            jax.ShapeDtypeStruct((N, D_H), F32),
            jax.ShapeDtypeStruct((N, D_CONV), F32),
            jax.ShapeDtypeStruct((N, D_CONV), F32),
            jax.ShapeDtypeStruct((N, 8), F32),
        ],
    )(y, na, a0, a1, sc, wl2, mp, wna, wl1, wmsg, wsc)


def _tc_fin(y, a0, a1, sc, wl2, m):
    def body(y_r, a0_r, a1_r, sc_r, wl2_r, m_r, x_r):
        yn = _gate_update(y_r[...], a0_r[...], a1_r[...], sc_r[...], wl2_r[...])
        x_r[...] = _dot(yn, m_r[...])

    return pl.pallas_call(
        body,
        grid=(N // _BN,),
        in_specs=[
            _rows(D_H), _rows(D_CONV), _rows(D_CONV), _rows(D_CONV),
            _full2d((D_CONV, D_CONV)), _full2d((D_H, D_IN)),
        ],
        out_specs=_rows(D_IN),
        out_shape=jax.ShapeDtypeStruct((N, D_IN), F32),
    )(y, a0, a1, sc, wl2, m)


def kernel(x, batch, node_attr, edge_src, edge_dst, K_PU, emb, W_na, W_lin1,
           W_fc1, b_fc1, W_fc2, W_sh, W_msg, W_lin2, W_sc):
    del batch
    G = K_PU.T @ K_PU
    w, V = jnp.linalg.eigh(G)
    G_isqrt = (V * (1.0 / jnp.sqrt(jnp.maximum(w, 1e-12)))) @ V.T
    M = K_PU @ G_isqrt

    mt = M.T
    xsel = jnp.pad(jnp.eye(3, dtype=F32), ((0, D_IN - 3), (0, 8 - 3)))
    mp = M @ xsel
    attr = node_attr.reshape(N, 1).astype(jnp.int32)
    bfc1 = b_fc1.reshape(LAYERS, 1, RH)

    y, na, y3, sc, xpad = _tc_pre(
        x, attr, emb, mt, xsel, W_na[0], W_lin1[0], W_msg[0], W_sc[0])

    for l in range(LAYERS):
        vs, vd = _sc_gather_pos(xpad, edge_src, edge_dst)
        wsh = _tc_edge(vs, vd, W_fc1[l], bfc1[l], W_fc2[l], W_sh[l])
        agg2 = _sc_scatter(y3, wsh, edge_src, edge_dst)
        a0 = agg2[0, :N]
        a1 = agg2[1, :N]
        if l + 1 < LAYERS:
            y, y3, sc, xpad = _tc_mid(y, na, a0, a1, sc, W_lin2[l], mp,
                                      W_na[l + 1], W_lin1[l + 1], W_msg[l + 1], W_sc[l + 1])
        else:
            xout = _tc_fin(y, a0, a1, sc, W_lin2[l], M)
    return xout
